# baseline (speedup 1.0000x reference)
import functools

import jax
import jax.numpy as jnp
from jax import lax
from jax.experimental import pallas as pl
from jax.experimental.pallas import tpu as pltpu
from jax.experimental.pallas import tpu_sc as plsc

B = 32
N = 64
D = 256
P = 4096

NC = 2
NS = 16
NW = NC * NS

CR = 64
NCHUNK = 2 * P // CR
NBUF = 4

_MESH = plsc.VectorSubcoreMesh(core_axis_name="c", subcore_axis_name="s")


@functools.partial(
    pl.kernel,
    mesh=_MESH,
    out_type=jax.ShapeDtypeStruct((B, 2 * P, D), jnp.float32),
    scratch_types=[
        pltpu.VMEM((NCHUNK, CR), jnp.int32),
        pltpu.VMEM((CR, D), jnp.float32),
        pltpu.VMEM((CR, D), jnp.float32),
        pltpu.VMEM((CR, D), jnp.float32),
        pltpu.VMEM((CR, D), jnp.float32),
        pltpu.SemaphoreType.DMA,
        pltpu.SemaphoreType.DMA,
        pltpu.SemaphoreType.DMA,
        pltpu.SemaphoreType.DMA,
        pltpu.SemaphoreType.DMA,
        pltpu.SemaphoreType.DMA,
        pltpu.SemaphoreType.DMA,
        pltpu.SemaphoreType.DMA,
    ],
)
def _pair_gather(table_hbm, ij_hbm, out_hbm, idx_v,
                 rows0, rows1, rows2, rows3,
                 g0, g1, g2, g3, s0, s1, s2, s3):
    w = lax.axis_index("s") * NC + lax.axis_index("c")
    base = w * N
    rows = (rows0, rows1, rows2, rows3)
    gsem = (g0, g1, g2, g3)
    ssem = (s0, s1, s2, s3)

    pltpu.sync_copy(ij_hbm, idx_v)

    def prep_body(ci, carry):
        for t in range(CR // 16):
            sl = pl.ds(t * 16, 16)
            idx_v[ci, sl] = idx_v[ci, sl] + base
        return carry

    lax.fori_loop(0, NCHUNK, prep_body, 0)

    def fire_gather(ci, b):
        pltpu.async_copy(table_hbm.at[idx_v.at[ci]], rows[b], gsem[b])

    def wait_gather(ci, b):
        pltpu.make_async_copy(
            table_hbm.at[idx_v.at[ci]], rows[b], gsem[b]).wait()

    def fire_scatter(ci, b):
        pltpu.async_copy(
            rows[b], out_hbm.at[w, pl.ds(ci * CR, CR)], ssem[b])

    def wait_scatter(ci, b):
        pltpu.make_async_copy(
            rows[b], out_hbm.at[w, pl.ds(ci * CR, CR)], ssem[b]).wait()

    for b in range(NBUF):
        fire_gather(b, b)

    def chunk_body(g, carry):
        for b in range(NBUF):
            ci = NBUF * g + b
            wait_gather(ci, b)
            fire_scatter(ci, b)
            pb = (b - 1) % NBUF

            @pl.when(jnp.logical_and(ci >= 1, ci + NBUF - 1 < NCHUNK))
            def _():
                wait_scatter(ci - 1, pb)
                fire_gather(ci + NBUF - 1, pb)
        return carry

    lax.fori_loop(0, NCHUNK // NBUF, chunk_body, 0)

    for k in range(NBUF):
        ci = NCHUNK - NBUF + k
        wait_scatter(ci, ci % NBUF)


def kernel(x, i, j):
    table = x.reshape(B * N, D)
    ij = jnp.stack([i, j], axis=1).reshape(NCHUNK, CR)
    return _pair_gather(table, ij).reshape(B, P, 2 * D)
